# baseline (speedup 1.0000x reference)
import functools

import jax
import jax.numpy as jnp
from jax import lax
from jax.experimental import pallas as pl
from jax.experimental.pallas import tpu as pltpu
from jax.experimental.pallas import tpu_sc as plsc

N_NODES = 10000
N_EDGES = 320000
D_FEAT = 128
LANES = 16

NUM_CORES = 2
NUM_SUBCORES = 16
NUM_WORKERS = NUM_CORES * NUM_SUBCORES
CHUNK = 128
N_CHUNKS = N_EDGES // CHUNK
NI = 80

_mesh = plsc.VectorSubcoreMesh(core_axis_name="c", subcore_axis_name="s")


@functools.partial(
    pl.kernel,
    out_type=jax.ShapeDtypeStruct((N_EDGES,), jnp.float32),
    mesh=_mesh,
    scratch_types=[
        [pltpu.VMEM((2, CHUNK), jnp.int32) for _ in range(4)],
        [pltpu.VMEM((CHUNK, D_FEAT), jnp.float32) for _ in range(2)],
        [pltpu.VMEM((CHUNK, D_FEAT), jnp.float32) for _ in range(2)],
        [pltpu.VMEM((CHUNK,), jnp.float32) for _ in range(2)],
        [pltpu.SemaphoreType.DMA for _ in range(4)],
        [pltpu.SemaphoreType.DMA for _ in range(2)],
        [pltpu.SemaphoreType.DMA for _ in range(2)],
    ],
)
def _dot_decoder(src_hbm, dst_hbm, eidx_hbm, out_hbm,
                 eidx, srows, drows, outc, sem_i, sem_g, sem_o):
    wid = lax.axis_index("s") * NUM_CORES + lax.axis_index("c")

    lane_ids = lax.iota(jnp.int32, LANES)
    gather_dnums = lax.GatherDimensionNumbers(
        offset_dims=(), collapsed_slice_dims=(0,), start_index_map=(0,))
    perm = {s: lane_ids ^ s for s in (8, 4, 2, 1)}
    lane_bit0 = {s: (lane_ids & s) == 0 for s in (8, 4, 2, 1)}

    def fold(v, s):
        return v + lax.gather(
            v, perm[s][:, None], gather_dnums, slice_sizes=(1,),
            mode=lax.GatherScatterMode.PROMISE_IN_BOUNDS)

    def merge_tree(vs):
        for s in (8, 4, 2, 1):
            n = len(vs) // 2
            vs = [jnp.where(lane_bit0[s], fold(vs[j], s), fold(vs[j + n], s))
                  for j in range(n)]
        return vs[0]

    def valid(i):
        return wid + i * NUM_WORKERS < N_CHUNKS

    def off_of(i):
        return (wid + i * NUM_WORKERS) * CHUNK

    def idx_start(i, q):
        @pl.when(valid(i))
        def _():
            pltpu.async_copy(
                eidx_hbm.at[:, pl.ds(off_of(i), CHUNK)], eidx[q], sem_i[q])

    def gather_start(i, q, b):
        @pl.when(valid(i))
        def _():
            pltpu.make_async_copy(
                eidx_hbm.at[:, pl.ds(off_of(i), CHUNK)], eidx[q],
                sem_i[q]).wait()
            pltpu.async_copy(src_hbm.at[eidx[q].at[0]], srows[b], sem_g[b])
            pltpu.async_copy(dst_hbm.at[eidx[q].at[1]], drows[b], sem_g[b])

    def consume(i, q, b):
        eb, sb, db, ob = eidx[q], srows[b], drows[b], outc[b]

        @pl.when(valid(i))
        def _():
            pltpu.make_async_copy(src_hbm.at[eb.at[0]], sb, sem_g[b]).wait()
            pltpu.make_async_copy(dst_hbm.at[eb.at[1]], db, sem_g[b]).wait()

            @pl.when(i >= 2)
            def _():
                pltpu.make_async_copy(
                    ob, out_hbm.at[pl.ds(off_of(i - 2), CHUNK)],
                    sem_o[b]).wait()

            @plsc.parallel_loop(0, CHUNK, step=LANES, unroll=2)
            def group_body(e0):
                vs = []
                for t in range(LANES):
                    prods = [sb[e0 + t, pl.ds(j * LANES, LANES)]
                             * db[e0 + t, pl.ds(j * LANES, LANES)]
                             for j in range(D_FEAT // LANES)]
                    while len(prods) > 1:
                        prods = [prods[2 * m] + prods[2 * m + 1]
                                 for m in range(len(prods) // 2)]
                    vs.append(prods[0])
                ob[pl.ds(e0, LANES)] = merge_tree(vs)
            pltpu.async_copy(ob, out_hbm.at[pl.ds(off_of(i), CHUNK)], sem_o[b])

    idx_start(0, 0)
    idx_start(1, 1)
    gather_start(0, 0, 0)

    def outer(i4, carry):
        i0 = i4 * 4
        for k in range(4):
            i = i0 + k
            idx_start(i + 2, (k + 2) % 4)
            gather_start(i + 1, (k + 1) % 4, (k + 1) % 2)
            consume(i, k, k % 2)
        return carry

    lax.fori_loop(0, NI // 4, outer, 0)

    for i in range(NI - 4, NI):
        @pl.when(valid(i) & ~valid(i + 2))
        def _(i=i):
            pltpu.make_async_copy(
                outc[i % 2], out_hbm.at[pl.ds(off_of(i), CHUNK)],
                sem_o[i % 2]).wait()


def kernel(src_node_embeddings, dst_node_embeddings, edge_index):
    return _dot_decoder(src_node_embeddings, dst_node_embeddings, edge_index)


# speedup vs baseline: 1.2084x; 1.2084x over previous
import functools

import jax
import jax.numpy as jnp
from jax import lax
from jax.experimental import pallas as pl
from jax.experimental.pallas import tpu as pltpu
from jax.experimental.pallas import tpu_sc as plsc

N_NODES = 10000
N_EDGES = 320000
D_FEAT = 128
LANES = 16

NUM_CORES = 2
NUM_SUBCORES = 16
NUM_WORKERS = NUM_CORES * NUM_SUBCORES
CHUNK = 128
N_CHUNKS = N_EDGES // CHUNK
NI = 80

_mesh = plsc.VectorSubcoreMesh(core_axis_name="c", subcore_axis_name="s")


@functools.partial(
    pl.kernel,
    out_type=jax.ShapeDtypeStruct((N_EDGES,), jnp.float32),
    mesh=_mesh,
    scratch_types=[
        [pltpu.VMEM((2, CHUNK), jnp.int32) for _ in range(4)],
        [pltpu.VMEM((CHUNK, D_FEAT), jnp.float32) for _ in range(2)],
        [pltpu.VMEM((CHUNK, D_FEAT), jnp.float32) for _ in range(2)],
        [pltpu.VMEM((CHUNK,), jnp.float32) for _ in range(2)],
        [pltpu.SemaphoreType.DMA for _ in range(4)],
        [pltpu.SemaphoreType.DMA for _ in range(2)],
        [pltpu.SemaphoreType.DMA for _ in range(2)],
    ],
)
def _dot_decoder(src_hbm, dst_hbm, eidx_hbm, out_hbm,
                 eidx, srows, drows, outc, sem_i, sem_g, sem_o):
    wid = lax.axis_index("s") * NUM_CORES + lax.axis_index("c")

    lane_ids = lax.iota(jnp.int32, LANES)
    gather_dnums = lax.GatherDimensionNumbers(
        offset_dims=(), collapsed_slice_dims=(0,), start_index_map=(0,))
    perm = {s: lane_ids ^ s for s in (8, 4, 2, 1)}
    lane_bit0 = {s: (lane_ids & s) == 0 for s in (8, 4, 2, 1)}

    def fold(v, s):
        return v + lax.gather(
            v, perm[s][:, None], gather_dnums, slice_sizes=(1,),
            mode=lax.GatherScatterMode.PROMISE_IN_BOUNDS)

    MERGE_SEQ = [0, 8, 4, 12, 2, 10, 6, 14, 1, 9, 5, 13, 3, 11, 7, 15]
    MERGE_S = (8, 4, 2, 1)

    def merge2(left, right, lvl):
        s = MERGE_S[lvl]
        return jnp.where(lane_bit0[s], fold(left, s), fold(right, s))

    def valid(i):
        return wid + i * NUM_WORKERS < N_CHUNKS

    def off_of(i):
        return (wid + i * NUM_WORKERS) * CHUNK

    def idx_start(i, q):
        @pl.when(valid(i))
        def _():
            pltpu.async_copy(
                eidx_hbm.at[:, pl.ds(off_of(i), CHUNK)], eidx[q], sem_i[q])

    def gather_start(i, q, b):
        @pl.when(valid(i))
        def _():
            pltpu.make_async_copy(
                eidx_hbm.at[:, pl.ds(off_of(i), CHUNK)], eidx[q],
                sem_i[q]).wait()
            pltpu.async_copy(src_hbm.at[eidx[q].at[0]], srows[b], sem_g[b])
            pltpu.async_copy(dst_hbm.at[eidx[q].at[1]], drows[b], sem_g[b])

    def consume(i, q, b):
        eb, sb, db, ob = eidx[q], srows[b], drows[b], outc[b]

        @pl.when(valid(i))
        def _():
            pltpu.make_async_copy(src_hbm.at[eb.at[0]], sb, sem_g[b]).wait()
            pltpu.make_async_copy(dst_hbm.at[eb.at[1]], db, sem_g[b]).wait()

            @pl.when(i >= 2)
            def _():
                pltpu.make_async_copy(
                    ob, out_hbm.at[pl.ds(off_of(i - 2), CHUNK)],
                    sem_o[b]).wait()

            @plsc.parallel_loop(0, CHUNK, step=LANES, unroll=1)
            def group_body(e0):
                stack = []
                for t in MERGE_SEQ:
                    prods = [sb[e0 + t, pl.ds(j * LANES, LANES)]
                             * db[e0 + t, pl.ds(j * LANES, LANES)]
                             for j in range(D_FEAT // LANES)]
                    while len(prods) > 1:
                        prods = [prods[2 * m] + prods[2 * m + 1]
                                 for m in range(len(prods) // 2)]
                    node = (0, prods[0])
                    while stack and stack[-1][0] == node[0]:
                        lvl, left = stack.pop()
                        node = (lvl + 1, merge2(left, node[1], lvl))
                    stack.append(node)
                ob[pl.ds(e0, LANES)] = stack[0][1]
            pltpu.async_copy(ob, out_hbm.at[pl.ds(off_of(i), CHUNK)], sem_o[b])

    idx_start(0, 0)
    idx_start(1, 1)
    gather_start(0, 0, 0)

    def outer(i4, carry):
        i0 = i4 * 4
        for k in range(4):
            i = i0 + k
            idx_start(i + 2, (k + 2) % 4)
            gather_start(i + 1, (k + 1) % 4, (k + 1) % 2)
            consume(i, k, k % 2)
        return carry

    lax.fori_loop(0, NI // 4, outer, 0)

    for i in range(NI - 4, NI):
        @pl.when(valid(i) & ~valid(i + 2))
        def _(i=i):
            pltpu.make_async_copy(
                outc[i % 2], out_hbm.at[pl.ds(off_of(i), CHUNK)],
                sem_o[i % 2]).wait()


def kernel(src_node_embeddings, dst_node_embeddings, edge_index):
    return _dot_decoder(src_node_embeddings, dst_node_embeddings, edge_index)


# speedup vs baseline: 1.3227x; 1.0945x over previous
import functools

import jax
import jax.numpy as jnp
from jax import lax
from jax.experimental import pallas as pl
from jax.experimental.pallas import tpu as pltpu
from jax.experimental.pallas import tpu_sc as plsc

N_NODES = 10000
N_EDGES = 320000
D_FEAT = 128
LANES = 16

NUM_CORES = 2
NUM_SUBCORES = 16
NUM_WORKERS = NUM_CORES * NUM_SUBCORES
CHUNK = 128
N_CHUNKS = N_EDGES // CHUNK
NI = 80

_mesh = plsc.VectorSubcoreMesh(core_axis_name="c", subcore_axis_name="s")


@functools.partial(
    pl.kernel,
    out_type=jax.ShapeDtypeStruct((N_EDGES,), jnp.float32),
    mesh=_mesh,
    scratch_types=[
        [pltpu.VMEM((2, CHUNK), jnp.int32) for _ in range(4)],
        [pltpu.VMEM((CHUNK, D_FEAT), jnp.float32) for _ in range(2)],
        [pltpu.VMEM((CHUNK, D_FEAT), jnp.float32) for _ in range(2)],
        [pltpu.VMEM((CHUNK,), jnp.float32) for _ in range(2)],
        [pltpu.SemaphoreType.DMA for _ in range(4)],
        [pltpu.SemaphoreType.DMA for _ in range(2)],
        [pltpu.SemaphoreType.DMA for _ in range(2)],
    ],
)
def _dot_decoder(src_hbm, dst_hbm, eidx_hbm, out_hbm,
                 eidx, srows, drows, outc, sem_i, sem_g, sem_o):
    wid = lax.axis_index("s") * NUM_CORES + lax.axis_index("c")

    lane_ids = lax.iota(jnp.int32, LANES)
    gather_dnums = lax.GatherDimensionNumbers(
        offset_dims=(), collapsed_slice_dims=(0,), start_index_map=(0,))
    perm = {s: lane_ids ^ s for s in (8, 4, 2, 1)}
    lane_bit0 = {s: (lane_ids & s) == 0 for s in (8, 4, 2, 1)}

    def fold(v, s):
        return v + lax.gather(
            v, perm[s][:, None], gather_dnums, slice_sizes=(1,),
            mode=lax.GatherScatterMode.PROMISE_IN_BOUNDS)

    MERGE_SEQ = [0, 8, 4, 12, 2, 10, 6, 14, 1, 9, 5, 13, 3, 11, 7, 15]
    MERGE_S = (8, 4, 2, 1)

    def merge2(left, right, lvl):
        s = MERGE_S[lvl]
        return jnp.where(lane_bit0[s], fold(left, s), fold(right, s))

    def valid(i):
        return wid + i * NUM_WORKERS < N_CHUNKS

    def off_of(i):
        return (wid + i * NUM_WORKERS) * CHUNK

    def idx_start(i, q):
        @pl.when(valid(i))
        def _():
            pltpu.async_copy(
                eidx_hbm.at[:, pl.ds(off_of(i), CHUNK)], eidx[q], sem_i[q])

    def gather_start(i, q, b):
        @pl.when(valid(i))
        def _():
            pltpu.make_async_copy(
                eidx_hbm.at[:, pl.ds(off_of(i), CHUNK)], eidx[q],
                sem_i[q]).wait()
            pltpu.async_copy(src_hbm.at[eidx[q].at[0]], srows[b], sem_g[b])
            pltpu.async_copy(dst_hbm.at[eidx[q].at[1]], drows[b], sem_g[b])

    def consume(i, q, b):
        eb, sb, db, ob = eidx[q], srows[b], drows[b], outc[b]

        @pl.when(valid(i))
        def _():
            pltpu.make_async_copy(src_hbm.at[eb.at[0]], sb, sem_g[b]).wait()
            pltpu.make_async_copy(dst_hbm.at[eb.at[1]], db, sem_g[b]).wait()

            @pl.when(i >= 2)
            def _():
                pltpu.make_async_copy(
                    ob, out_hbm.at[pl.ds(off_of(i - 2), CHUNK)],
                    sem_o[b]).wait()

            @plsc.parallel_loop(0, CHUNK, step=LANES, unroll=1)
            def group_body(e0):
                stack = []
                for t in MERGE_SEQ:
                    e = e0 + t
                    acc = sb[e, pl.ds(0, LANES)] * db[e, pl.ds(0, LANES)]
                    for j in range(1, D_FEAT // LANES):
                        acc = acc + (sb[e, pl.ds(j * LANES, LANES)]
                                     * db[e, pl.ds(j * LANES, LANES)])
                    node = (0, acc)
                    while stack and stack[-1][0] == node[0]:
                        lvl, left = stack.pop()
                        node = (lvl + 1, merge2(left, node[1], lvl))
                    stack.append(node)
                ob[pl.ds(e0, LANES)] = stack[0][1]
            pltpu.async_copy(ob, out_hbm.at[pl.ds(off_of(i), CHUNK)], sem_o[b])

    idx_start(0, 0)
    idx_start(1, 1)
    gather_start(0, 0, 0)

    def outer(i4, carry):
        i0 = i4 * 4
        for k in range(4):
            i = i0 + k
            idx_start(i + 2, (k + 2) % 4)
            gather_start(i + 1, (k + 1) % 4, (k + 1) % 2)
            consume(i, k, k % 2)
        return carry

    lax.fori_loop(0, NI // 4, outer, 0)

    for i in range(NI - 4, NI):
        @pl.when(valid(i) & ~valid(i + 2))
        def _(i=i):
            pltpu.make_async_copy(
                outc[i % 2], out_hbm.at[pl.ds(off_of(i), CHUNK)],
                sem_o[i % 2]).wait()


def kernel(src_node_embeddings, dst_node_embeddings, edge_index):
    return _dot_decoder(src_node_embeddings, dst_node_embeddings, edge_index)


# speedup vs baseline: 2.2324x; 1.6877x over previous
import functools

import jax
import jax.numpy as jnp
from jax import lax
from jax.experimental import pallas as pl
from jax.experimental.pallas import tpu as pltpu
from jax.experimental.pallas import tpu_sc as plsc

N_NODES = 10000
N_EDGES = 320000
D_FEAT = 128
LANES = 16

NUM_CORES = 2
NUM_SUBCORES = 16
NUM_WORKERS = NUM_CORES * NUM_SUBCORES
CHUNK = 128
N_CHUNKS = N_EDGES // CHUNK
NI = 80

_mesh = plsc.VectorSubcoreMesh(core_axis_name="c", subcore_axis_name="s")


@functools.partial(
    pl.kernel,
    out_type=jax.ShapeDtypeStruct((N_EDGES,), jnp.float32),
    mesh=_mesh,
    scratch_types=[
        [pltpu.VMEM((2, CHUNK), jnp.int32) for _ in range(4)],
        [pltpu.VMEM((CHUNK, D_FEAT), jnp.float32) for _ in range(2)],
        [pltpu.VMEM((CHUNK, D_FEAT), jnp.float32) for _ in range(2)],
        [pltpu.VMEM((CHUNK,), jnp.float32) for _ in range(2)],
        pltpu.VMEM((CHUNK, LANES), jnp.float32),
        [pltpu.SemaphoreType.DMA for _ in range(4)],
        [pltpu.SemaphoreType.DMA for _ in range(2)],
        [pltpu.SemaphoreType.DMA for _ in range(2)],
    ],
)
def _dot_decoder(src_hbm, dst_hbm, eidx_hbm, out_hbm,
                 eidx, srows, drows, outc, accbuf, sem_i, sem_g, sem_o):
    wid = lax.axis_index("s") * NUM_CORES + lax.axis_index("c")

    lane_ids = lax.iota(jnp.int32, LANES)
    gather_dnums = lax.GatherDimensionNumbers(
        offset_dims=(), collapsed_slice_dims=(0,), start_index_map=(0,))
    perm = {s: lane_ids ^ s for s in (8, 4, 2, 1)}
    lane_bit0 = {s: (lane_ids & s) == 0 for s in (8, 4, 2, 1)}

    def fold(v, s):
        return v + lax.gather(
            v, perm[s][:, None], gather_dnums, slice_sizes=(1,),
            mode=lax.GatherScatterMode.PROMISE_IN_BOUNDS)

    MERGE_SEQ = [0, 8, 4, 12, 2, 10, 6, 14, 1, 9, 5, 13, 3, 11, 7, 15]
    MERGE_S = (8, 4, 2, 1)

    def merge2(left, right, lvl):
        s = MERGE_S[lvl]
        return jnp.where(lane_bit0[s], fold(left, s), fold(right, s))

    def valid(i):
        return wid + i * NUM_WORKERS < N_CHUNKS

    def off_of(i):
        return (wid + i * NUM_WORKERS) * CHUNK

    def idx_start(i, q):
        @pl.when(valid(i))
        def _():
            pltpu.async_copy(
                eidx_hbm.at[:, pl.ds(off_of(i), CHUNK)], eidx[q], sem_i[q])

    def gather_start(i, q, b):
        @pl.when(valid(i))
        def _():
            pltpu.make_async_copy(
                eidx_hbm.at[:, pl.ds(off_of(i), CHUNK)], eidx[q],
                sem_i[q]).wait()
            pltpu.async_copy(src_hbm.at[eidx[q].at[0]], srows[b], sem_g[b])
            pltpu.async_copy(dst_hbm.at[eidx[q].at[1]], drows[b], sem_g[b])

    def consume(i, q, b):
        eb, sb, db, ob = eidx[q], srows[b], drows[b], outc[b]

        @pl.when(valid(i))
        def _():
            pltpu.make_async_copy(src_hbm.at[eb.at[0]], sb, sem_g[b]).wait()
            pltpu.make_async_copy(dst_hbm.at[eb.at[1]], db, sem_g[b]).wait()

            @pl.when(i >= 2)
            def _():
                pltpu.make_async_copy(
                    ob, out_hbm.at[pl.ds(off_of(i - 2), CHUNK)],
                    sem_o[b]).wait()

            @plsc.parallel_loop(0, CHUNK, step=1, unroll=2)
            def edge_acc(e):
                acc = sb[e, pl.ds(0, LANES)] * db[e, pl.ds(0, LANES)]
                for j in range(1, D_FEAT // LANES):
                    acc = acc + (sb[e, pl.ds(j * LANES, LANES)]
                                 * db[e, pl.ds(j * LANES, LANES)])
                accbuf[e, pl.ds(0, LANES)] = acc

            @plsc.parallel_loop(0, CHUNK, step=LANES, unroll=1)
            def group_merge(e0):
                stack = []
                for t in MERGE_SEQ:
                    node = (0, accbuf[e0 + t, pl.ds(0, LANES)])
                    while stack and stack[-1][0] == node[0]:
                        lvl, left = stack.pop()
                        node = (lvl + 1, merge2(left, node[1], lvl))
                    stack.append(node)
                ob[pl.ds(e0, LANES)] = stack[0][1]
            pltpu.async_copy(ob, out_hbm.at[pl.ds(off_of(i), CHUNK)], sem_o[b])

    idx_start(0, 0)
    idx_start(1, 1)
    gather_start(0, 0, 0)

    def outer(i4, carry):
        i0 = i4 * 4
        for k in range(4):
            i = i0 + k
            idx_start(i + 2, (k + 2) % 4)
            gather_start(i + 1, (k + 1) % 4, (k + 1) % 2)
            consume(i, k, k % 2)
        return carry

    lax.fori_loop(0, NI // 4, outer, 0)

    for i in range(NI - 4, NI):
        @pl.when(valid(i) & ~valid(i + 2))
        def _(i=i):
            pltpu.make_async_copy(
                outc[i % 2], out_hbm.at[pl.ds(off_of(i), CHUNK)],
                sem_o[i % 2]).wait()


def kernel(src_node_embeddings, dst_node_embeddings, edge_index):
    return _dot_decoder(src_node_embeddings, dst_node_embeddings, edge_index)
